# baseline (speedup 1.0000x reference)
import jax
import jax.numpy as jnp
from jax.experimental import pallas as pl
from jax.experimental.pallas import tpu as pltpu

F = 64
NX = 512
NY = 512
C = NX * NY
P = 30000

ZCHUNK = 16384
BATCH = 64


def _scatter_body(coords_ref, pf_ref, out_ref, zbuf_ref, sem, rsem):
    zbuf_ref[...] = jnp.zeros_like(zbuf_ref)
    nz = C // ZCHUNK
    for k in range(nz):
        pltpu.make_async_copy(
            zbuf_ref, out_ref.at[pl.ds(k * ZCHUNK, ZCHUNK), :], sem
        ).start()
    for k in range(nz):
        pltpu.make_async_copy(
            zbuf_ref, out_ref.at[pl.ds(k * ZCHUNK, ZCHUNK), :], sem
        ).wait()

    def flat_idx(p):
        return (coords_ref[4 * p + 1]
                + coords_ref[4 * p + 2] * NX
                + coords_ref[4 * p + 3])

    def group(g, carry):
        base = g * BATCH
        for j in range(BATCH):
            p = base + j
            i = flat_idx(p)
            pltpu.make_async_copy(
                pf_ref.at[pl.ds(p, 1), :], out_ref.at[pl.ds(i, 1), :], rsem
            ).start()
        for j in range(BATCH):
            p = base + j
            i = flat_idx(p)
            pltpu.make_async_copy(
                pf_ref.at[pl.ds(p, 1), :], out_ref.at[pl.ds(i, 1), :], rsem
            ).wait()
        return carry

    jax.lax.fori_loop(0, P // BATCH, group, 0)
    for j in range(P % BATCH):
        p = (P // BATCH) * BATCH + j
        i = flat_idx(p)
        pltpu.make_async_copy(
            pf_ref.at[pl.ds(p, 1), :], out_ref.at[pl.ds(i, 1), :], rsem
        ).start()
        pltpu.make_async_copy(
            pf_ref.at[pl.ds(p, 1), :], out_ref.at[pl.ds(i, 1), :], rsem
        ).wait()


def _transpose_body(in_ref, out_ref):
    out_ref[...] = in_ref[...].T


TB = 2048


def kernel(pillar_features, coords):
    coords_flat = coords.reshape(-1).astype(jnp.int32)

    grid_spec = pltpu.PrefetchScalarGridSpec(
        num_scalar_prefetch=1,
        grid=(1,),
        in_specs=[pl.BlockSpec((P, F), lambda i, c: (0, 0))],
        out_specs=pl.BlockSpec(memory_space=pl.ANY),
        scratch_shapes=[
            pltpu.VMEM((ZCHUNK, F), jnp.float32),
            pltpu.SemaphoreType.DMA,
            pltpu.SemaphoreType.DMA,
        ],
    )
    scratch = pl.pallas_call(
        _scatter_body,
        grid_spec=grid_spec,
        out_shape=jax.ShapeDtypeStruct((C, F), jnp.float32),
    )(coords_flat, pillar_features)

    out = pl.pallas_call(
        _transpose_body,
        grid=(C // TB,),
        in_specs=[pl.BlockSpec((TB, F), lambda i: (i, 0))],
        out_specs=pl.BlockSpec((F, TB), lambda i: (0, i)),
        out_shape=jax.ShapeDtypeStruct((F, C), jnp.float32),
    )(scratch)
    return out.reshape(1, F, NY, NX)
